# baseline (speedup 1.0000x reference)
import jax
import jax.numpy as jnp
from jax import lax
from jax.experimental import pallas as pl
from jax.experimental.pallas import tpu as pltpu
from jax.experimental.pallas import tpu_sc as plsc

D = 64
L = 16
NC, NS = 2, 16
NW = NC * NS
CHUNK = 128
COFF = 0.001

GCOLS = (0, 1, 2, 3, 4, 6, 7)
NG = len(GCOLS)


def _sc_body(x_hbm, cols_hbm, out_hbm, *refs):
    idx_v = refs[0:9]
    rows_v = refs[9]
    out_v = refs[10:13]
    tb = refs[13:16]
    sem = refs[16]
    B = cols_hbm.shape[0] // 9
    pw = B // NW
    nchunk = pw // CHUNK
    ngroup = CHUNK // L

    wid = lax.axis_index("s") * NC + lax.axis_index("c")
    base = wid * pw

    for j in range(9):
        pltpu.sync_copy(cols_hbm.at[pl.ds(j * B + base, pw)], idx_v[j])

    lanes16 = lax.iota(jnp.int32, L) * L

    def chunk_body(c, carry):
        copies = []
        for slot, col in enumerate(GCOLS):
            copies.append(pltpu.async_copy(
                x_hbm.at[idx_v[col].at[pl.ds(c * CHUNK, CHUNK)]],
                rows_v.at[pl.ds(slot * CHUNK, CHUNK)],
                sem))
        for cp in copies:
            cp.wait()

        def group_body(g, carry2):
            p0 = g * L
            for j in range(L):
                row = p0 + j
                u = [rows_v[0 * CHUNK + row, pl.ds(k * L, L)] for k in range(D // L)]
                ip = [rows_v[1 * CHUNK + row, pl.ds(k * L, L)] for k in range(D // L)]
                inn = [rows_v[2 * CHUNK + row, pl.ds(k * L, L)] for k in range(D // L)]
                eip = [rows_v[3 * CHUNK + row, pl.ds(k * L, L)] for k in range(D // L)]
                ein = [rows_v[4 * CHUNK + row, pl.ds(k * L, L)] for k in range(D // L)]
                eup = [rows_v[5 * CHUNK + row, pl.ds(k * L, L)] for k in range(D // L)]
                eun = [rows_v[6 * CHUNK + row, pl.ds(k * L, L)] for k in range(D // L)]
                vcf = vi = vu = None
                for k in range(D // L):
                    tcf = u[k] * (ip[k] - inn[k])
                    a = ip[k] - eip[k]
                    b = ip[k] - ein[k]
                    ti = a * a - b * b
                    a = u[k] - eup[k]
                    b = u[k] - eun[k]
                    tu = a * a - b * b
                    vcf = tcf if vcf is None else vcf + tcf
                    vi = ti if vi is None else vi + ti
                    vu = tu if vu is None else vu + tu
                sidx = lanes16 + j
                plsc.store_scatter(tb[0], [sidx], vcf)
                plsc.store_scatter(tb[1], [sidx], vi)
                plsc.store_scatter(tb[2], [sidx], vu)
            zs = []
            for t in range(3):
                acc = tb[t][pl.ds(0, L)]
                for l in range(1, L):
                    acc = acc + tb[t][pl.ds(l * L, L)]
                zs.append(acc)
            zcf, zi, zu = zs
            off = c * CHUNK + g * L
            mi = idx_v[5][pl.ds(off, L)].astype(jnp.float32)
            mu = idx_v[8][pl.ds(off, L)].astype(jnp.float32)
            out_v[0][pl.ds(off, L)] = zcf
            out_v[1][pl.ds(off, L)] = zi * mi
            out_v[2][pl.ds(off, L)] = zu * mu
            return carry2

        return lax.fori_loop(0, ngroup, group_body, carry)

    lax.fori_loop(0, nchunk, chunk_body, 0)
    for r in range(3):
        pltpu.sync_copy(out_v[r], out_hbm.at[pl.ds(r * B + base, pw)])


def _sc_pairs(x, cols):
    B = cols.shape[0] // 9
    mesh = plsc.VectorSubcoreMesh(
        core_axis_name="c", subcore_axis_name="s",
        num_cores=NC, num_subcores=NS)
    kfn = pl.kernel(
        _sc_body,
        out_type=jax.ShapeDtypeStruct((3 * B,), jnp.float32),
        mesh=mesh,
        compiler_params=pltpu.CompilerParams(needs_layout_passes=False, use_tc_tiling_on_sc=False),
        scratch_types=(
            [pltpu.VMEM((B // NW,), jnp.int32)] * 9
            + [pltpu.VMEM((NG * CHUNK, D), jnp.float32)]
            + [pltpu.VMEM((B // NW,), jnp.float32)] * 3
            + [pltpu.VMEM((L * L,), jnp.float32)] * 3
            + [pltpu.SemaphoreType.DMA]
        ),
    )
    return kfn(x, cols)


def _loss_body(z_ref, o_ref):
    z = z_ref[...]
    ls = jnp.minimum(z, 0.0) - jnp.log1p(jnp.exp(-jnp.abs(z)))
    total = -(jnp.sum(ls[0, :])
              + COFF * (jnp.sum(ls[1, :]) + jnp.sum(ls[2, :])))
    o_ref[...] = jnp.reshape(total, (1, 1))


def kernel(x, pos_neg_pair_t):
    B = pos_neg_pair_t.shape[0]
    cols = pos_neg_pair_t.astype(jnp.int32).T.reshape(-1)
    z = _sc_pairs(x, cols).reshape(3, B)
    loss2d = pl.pallas_call(
        _loss_body,
        out_shape=jax.ShapeDtypeStruct((1, 1), jnp.float32),
    )(z)
    return loss2d[0, 0]
